# baseline (speedup 1.0000x reference)
import functools

import jax
import jax.numpy as jnp
from jax import lax
from jax.experimental import pallas as pl
from jax.experimental.pallas import tpu as pltpu
from jax.experimental.pallas import tpu_sc as plsc

_LANES = 16
_NC = 2
_NS = 16
_CH = 32
_RING = 3


def kernel(input_ids, wte, wpe):
    b, s = input_ids.shape
    _, e = wte.shape
    n = b * s
    ids_flat = input_ids.reshape(n).astype(jnp.int32)

    nunits = _NC * _NS
    ppu = s // nunits
    nchunks = ppu // _CH
    nitems = b * nchunks

    mesh = plsc.VectorSubcoreMesh(core_axis_name="c", subcore_axis_name="s")

    scratch = (
        [pltpu.VMEM((b * ppu,), jnp.int32)]
        + [pltpu.VMEM((ppu, e), jnp.float32)]
        + [pltpu.VMEM((_CH, e), jnp.float32) for _ in range(_RING)]
        + [pltpu.SemaphoreType.DMA for _ in range(1 + b + 2 * _RING)]
    )

    @functools.partial(
        pl.kernel,
        out_type=jax.ShapeDtypeStruct((n, e), jnp.float32),
        mesh=mesh,
        scratch_types=scratch,
    )
    def run(wte_hbm, ids_hbm, wpe_hbm, out_hbm, ids_v, wpe_v, *rest):
        rows = rest[:_RING]
        sem_wpe = rest[_RING]
        sem_ids = rest[_RING + 1:_RING + 1 + b]
        sem_g = rest[_RING + 1 + b:_RING + 1 + b + _RING]
        sem_o = rest[_RING + 1 + b + _RING:]

        wid = lax.axis_index("s") * _NC + lax.axis_index("c")
        pos0 = wid * ppu

        h_ids = [
            pltpu.async_copy(
                ids_hbm.at[pl.ds(bb * s + pos0, ppu)],
                ids_v.at[pl.ds(bb * ppu, ppu)],
                sem_ids[bb],
            )
            for bb in range(b)
        ]
        h_wpe = pltpu.async_copy(wpe_hbm.at[pl.ds(pos0, ppu)], wpe_v, sem_wpe)
        ids_ready = [False] * b

        def ensure_ids(j):
            bb = (j * _CH) // ppu
            if not ids_ready[bb]:
                h_ids[bb].wait()
                ids_ready[bb] = True

        def fire_gather(j):
            return pltpu.async_copy(
                wte_hbm.at[ids_v.at[pl.ds(j * _CH, _CH)]],
                rows[j % _RING],
                sem_g[j % _RING],
            )

        hg = {}
        ho = {}
        for j in range(_RING - 1):
            ensure_ids(j)
            hg[j] = fire_gather(j)
        h_wpe.wait()

        for j in range(nitems):
            bb, c = divmod(j, nchunks)
            buf = rows[j % _RING]
            with jax.named_scope("gwait"):
                hg[j].wait()

            with jax.named_scope("add"):
                @plsc.parallel_loop(0, _CH, unroll=2)
                def _(r):
                    for cc in range(0, e, _LANES):
                        slc = (pl.ds(r, 1), pl.ds(cc, _LANES))
                        wslc = (pl.ds(c * _CH + r, 1), pl.ds(cc, _LANES))
                        plsc.addupdate(buf.at[slc], wpe_v.at[wslc][...])

            ho[j] = pltpu.async_copy(
                buf,
                out_hbm.at[pl.ds(bb * s + pos0 + c * _CH, _CH)],
                sem_o[j % _RING],
            )
            nxt = j + _RING - 1
            if nxt < nitems:
                if j >= 1:
                    with jax.named_scope("owait"):
                        ho[j - 1].wait()
                ensure_ids(nxt)
                hg[nxt] = fire_gather(nxt)

        for j in range(max(0, nitems - _RING), nitems):
            if j in ho:
                ho[j].wait()

    out = run(wte, ids_flat, wpe)
    return out.reshape(b, s, e)


# speedup vs baseline: 1.0665x; 1.0665x over previous
import functools

import jax
import jax.numpy as jnp
from jax import lax
from jax.experimental import pallas as pl
from jax.experimental.pallas import tpu as pltpu
from jax.experimental.pallas import tpu_sc as plsc

_LANES = 16
_NC = 2
_NS = 16
_CH = 16
_PAIR = 2
_RING = 3


def kernel(input_ids, wte, wpe):
    b, s = input_ids.shape
    _, e = wte.shape
    n = b * s
    ids_flat = input_ids.reshape(n).astype(jnp.int32)

    nunits = _NC * _NS
    ppu = s // nunits
    nchunks = ppu // _CH
    npairs = b // _PAIR
    ngroups = npairs * nchunks
    nbufs = _RING * _PAIR

    mesh = plsc.VectorSubcoreMesh(core_axis_name="c", subcore_axis_name="s")

    scratch = (
        [pltpu.VMEM((b * ppu,), jnp.int32)]
        + [pltpu.VMEM((ppu, e), jnp.float32)]
        + [pltpu.VMEM((_CH, e), jnp.float32) for _ in range(nbufs)]
        + [pltpu.SemaphoreType.DMA for _ in range(1 + b + 2 * nbufs)]
    )

    @functools.partial(
        pl.kernel,
        out_type=jax.ShapeDtypeStruct((n, e), jnp.float32),
        mesh=mesh,
        scratch_types=scratch,
    )
    def run(wte_hbm, ids_hbm, wpe_hbm, out_hbm, ids_v, wpe_v, *rest):
        rows = rest[:nbufs]
        sem_wpe = rest[nbufs]
        sem_ids = rest[nbufs + 1:nbufs + 1 + b]
        sem_g = rest[nbufs + 1 + b:nbufs + 1 + b + nbufs]
        sem_o = rest[nbufs + 1 + b + nbufs:]

        wid = lax.axis_index("s") * _NC + lax.axis_index("c")
        pos0 = wid * ppu

        h_ids = [
            pltpu.async_copy(
                ids_hbm.at[pl.ds(bb * s + pos0, ppu)],
                ids_v.at[pl.ds(bb * ppu, ppu)],
                sem_ids[bb],
            )
            for bb in range(b)
        ]
        h_wpe = pltpu.async_copy(wpe_hbm.at[pl.ds(pos0, ppu)], wpe_v, sem_wpe)
        ids_ready = [False] * b

        def bufs_of(g):
            k = g % _RING
            return rows[k * _PAIR:(k + 1) * _PAIR]

        def group_rows(g):
            p, c = divmod(g, nchunks)
            return [p * _PAIR + k for k in range(_PAIR)], c

        def fire_gathers(g):
            brs, c = group_rows(g)
            hs = []
            for k, bb in enumerate(brs):
                if not ids_ready[bb]:
                    h_ids[bb].wait()
                    ids_ready[bb] = True
                hs.append(pltpu.async_copy(
                    wte_hbm.at[ids_v.at[pl.ds(bb * ppu + c * _CH, _CH)]],
                    bufs_of(g)[k],
                    sem_g[(g % _RING) * _PAIR + k],
                ))
            return hs

        hg, ho = {}, {}
        for g in range(_RING - 1):
            hg[g] = fire_gathers(g)
        h_wpe.wait()

        for g in range(ngroups):
            brs, c = group_rows(g)
            b0, b1 = bufs_of(g)
            with jax.named_scope("gwait"):
                for h in hg[g]:
                    h.wait()

            with jax.named_scope("add"):
                @plsc.parallel_loop(0, _CH)
                def _(r):
                    for cc in range(0, e, _LANES):
                        slc = (pl.ds(r, 1), pl.ds(cc, _LANES))
                        w = wpe_v.at[pl.ds(c * _CH + r, 1), pl.ds(cc, _LANES)][...]
                        plsc.addupdate(b0.at[slc], w)
                        plsc.addupdate(b1.at[slc], w)

            ho[g] = [
                pltpu.async_copy(
                    bufs_of(g)[k],
                    out_hbm.at[pl.ds(bb * s + pos0 + c * _CH, _CH)],
                    sem_o[(g % _RING) * _PAIR + k],
                )
                for k, bb in enumerate(brs)
            ]
            nxt = g + _RING - 1
            if nxt < ngroups:
                if g >= 1:
                    with jax.named_scope("owait"):
                        for h in ho[g - 1]:
                            h.wait()
                hg[nxt] = fire_gathers(nxt)

        for g in range(max(0, ngroups - _RING), ngroups):
            if g in ho:
                for h in ho[g]:
                    h.wait()

    out = run(wte, ids_flat, wpe)
    return out.reshape(b, s, e)
